# baseline (speedup 1.0000x reference)
import functools

import jax
import jax.numpy as jnp
from jax import lax
from jax.experimental import pallas as pl
from jax.experimental.pallas import tpu as pltpu
from jax.experimental.pallas import tpu_sc as plsc

C = 64
P = 20000
NXG = 512
S = 262144
P_PAD = 20016
IDX_PAD = 20096
SENT_IDX = 1 << 22
NC, NS = 2, 16
NW = NC * NS
CELLS = S // NW
G_PIL = IDX_PAD // 16
G_CELL = CELLS // 16


def _prep_body(feat_ref, coords_ref, tbl_ref, idx_ref):
    x = feat_ref[...]
    xp = jnp.concatenate(
        [x, jnp.zeros((P_PAD - P, C), jnp.float32)], axis=0)
    tbl_ref[...] = xp.T
    cr = coords_ref[...]
    iv = cr[:, 1] + cr[:, 2] * NXG + cr[:, 3]
    idx_ref[...] = jnp.concatenate(
        [iv, jnp.full((IDX_PAD - P,), SENT_IDX, jnp.int32)]).reshape(
            IDX_PAD // 128, 128)


_prep = pl.pallas_call(
    _prep_body,
    compiler_params=pltpu.CompilerParams(vmem_limit_bytes=100 * 2**20),
    out_shape=(
        jax.ShapeDtypeStruct((C, P_PAD), jnp.float32),
        jax.ShapeDtypeStruct((IDX_PAD // 128, 128), jnp.int32),
    ),
)

_mesh = plsc.VectorSubcoreMesh(
    core_axis_name="c", subcore_axis_name="s", num_cores=NC, num_subcores=NS)


@functools.partial(
    pl.kernel,
    mesh=_mesh,
    compiler_params=pltpu.CompilerParams(needs_layout_passes=False),
    out_type=jax.ShapeDtypeStruct((C * S,), jnp.float32),
    scratch_types=[
        pltpu.VMEM((IDX_PAD,), jnp.int32),
        pltpu.VMEM((CELLS,), jnp.int32),
        pltpu.VMEM((P_PAD,), jnp.float32),
        pltpu.VMEM((P_PAD,), jnp.float32),
        pltpu.VMEM((CELLS,), jnp.float32),
        pltpu.VMEM((CELLS,), jnp.float32),
    ],
)
def _sc_scatter(tbl_hbm, idx_hbm, out_hbm, idx_v, inv_v, t0, t1, o0, o1):
    wid = lax.axis_index("s") * NC + lax.axis_index("c")
    lo = wid * CELLS

    pltpu.sync_copy(idx_hbm, idx_v)

    sent = jnp.full((16,), P, jnp.int32)

    def init_body(g, carry):
        inv_v[pl.ds(g * 16, 16)] = sent
        return carry

    lax.fori_loop(0, G_CELL, init_body, 0)

    lane = lax.broadcasted_iota(jnp.int32, (16,), 0)

    def scan_body(g, carry):
        rel = idx_v[pl.ds(g * 16, 16)] - lo
        m = (rel >= 0) & (rel < CELLS)
        rel_safe = jnp.where(m, rel, 0)
        plsc.store_scatter(inv_v, [rel_safe], g * 16 + lane, mask=m)
        return carry

    lax.fori_loop(0, G_PIL, scan_body, 0)

    for cb in range(C // 2):
        c0 = 2 * cb
        pltpu.sync_copy(tbl_hbm.at[pl.ds(c0 * P_PAD, P_PAD)], t0)
        pltpu.sync_copy(tbl_hbm.at[pl.ds((c0 + 1) * P_PAD, P_PAD)], t1)

        def gat_body(g, carry):
            iv = inv_v[pl.ds(g * 16, 16)]
            o0[pl.ds(g * 16, 16)] = plsc.load_gather(t0, [iv])
            o1[pl.ds(g * 16, 16)] = plsc.load_gather(t1, [iv])
            return carry

        lax.fori_loop(0, G_CELL, gat_body, 0)
        pltpu.sync_copy(o0, out_hbm.at[pl.ds(c0 * S + lo, CELLS)])
        pltpu.sync_copy(o1, out_hbm.at[pl.ds((c0 + 1) * S + lo, CELLS)])


def kernel(pillar_features, coords):
    tbl, idx2 = _prep(pillar_features, coords)
    out = _sc_scatter(tbl.reshape(-1), idx2.reshape(-1))
    return out.reshape(1, C, NXG, NXG)


# speedup vs baseline: 1.7017x; 1.7017x over previous
import functools

import jax
import jax.numpy as jnp
from jax import lax
from jax.experimental import pallas as pl
from jax.experimental.pallas import tpu as pltpu
from jax.experimental.pallas import tpu_sc as plsc

C = 64
P = 20000
NXG = 512
S = 262144
P_PAD = 20016
IDX_PAD = 20480
SENT_IDX = 1 << 22
NC, NS = 2, 16
NW = NC * NS
CELLS = S // NW
G_CELL = CELLS // 16
ICHUNK = 4096
NCHUNK = IDX_PAD // ICHUNK
IROWS = ICHUNK // 128


def _prep_body(feat_ref, coords_ref, tbl_ref, idx_ref):
    x = feat_ref[...]
    xp = jnp.concatenate(
        [x, jnp.zeros((P_PAD - P, C), jnp.float32)], axis=0)
    tbl_ref[...] = xp.T
    cr = coords_ref[...]
    iv = cr[:, 1] + cr[:, 2] * NXG + cr[:, 3]
    idx_ref[...] = jnp.concatenate(
        [iv, jnp.full((IDX_PAD - P,), SENT_IDX, jnp.int32)]).reshape(
            IDX_PAD // 128, 128)


_prep = pl.pallas_call(
    _prep_body,
    compiler_params=pltpu.CompilerParams(vmem_limit_bytes=100 * 2**20),
    out_shape=(
        jax.ShapeDtypeStruct((C, P_PAD), jnp.float32),
        jax.ShapeDtypeStruct((IDX_PAD // 128, 128), jnp.int32),
    ),
)

_mesh = plsc.VectorSubcoreMesh(
    core_axis_name="c", subcore_axis_name="s", num_cores=NC, num_subcores=NS)


@functools.partial(
    pl.kernel,
    mesh=_mesh,
    compiler_params=pltpu.CompilerParams(needs_layout_passes=False),
    out_type=jax.ShapeDtypeStruct((C * S,), jnp.float32),
    scratch_types=[
        [pltpu.VMEM((IROWS, 128), jnp.int32)] * 2,
        pltpu.VMEM((CELLS,), jnp.int32),
        [pltpu.VMEM((P_PAD,), jnp.float32)] * 4,
        [pltpu.VMEM((CELLS,), jnp.float32)] * 4,
        [pltpu.SemaphoreType.DMA] * 2,
        [pltpu.SemaphoreType.DMA] * 4,
        [pltpu.SemaphoreType.DMA] * 4,
    ],
)
def _sc_scatter(tbl_hbm, idx_hbm, out_hbm, ib, inv_v, tb, ob, isem, tsem, osem):
    wid = lax.axis_index("s") * NC + lax.axis_index("c")
    lo = wid * CELLS

    tdesc = [
        pltpu.async_copy(tbl_hbm.at[c], tb[c], tsem[c]) for c in (0, 1)
    ]
    idesc = pltpu.async_copy(idx_hbm.at[pl.ds(0, IROWS)], ib[0], isem[0])

    sent = jnp.full((16,), P, jnp.int32)

    @plsc.parallel_loop(0, CELLS, 16, unroll=8)
    def _init(i):
        inv_v[pl.ds(i, 16)] = sent

    lane = lax.broadcasted_iota(jnp.int32, (16,), 0)
    for ch in range(NCHUNK):
        nxt = pltpu.async_copy(
            idx_hbm.at[pl.ds((ch + 1) * IROWS, IROWS)],
            ib[(ch + 1) % 2], isem[(ch + 1) % 2]) if ch + 1 < NCHUNK else None
        idesc.wait()
        buf = ib[ch % 2]
        base = ch * ICHUNK

        def scan_body(g, carry, buf=buf, base=base):
            rel = buf[g // 8, pl.ds((g % 8) * 16, 16)] - lo
            m = (rel >= 0) & (rel < CELLS)
            rel_safe = jnp.where(m, rel, 0)
            plsc.store_scatter(
                inv_v, [rel_safe], base + g * 16 + lane, mask=m)
            return carry

        lax.fori_loop(0, ICHUNK // 16, scan_body, 0)
        idesc = nxt

    odesc = [None] * 4
    for k in range(C // 2):
        cur, nxt = 2 * (k % 2), 2 * ((k + 1) % 2)
        if k + 1 < C // 2:
            tnext = [
                pltpu.async_copy(
                    tbl_hbm.at[2 * (k + 1) + j], tb[nxt + j], tsem[nxt + j])
                for j in (0, 1)
            ]
        else:
            tnext = None
        tdesc[0].wait()
        tdesc[1].wait()
        for j in (0, 1):
            if odesc[cur + j] is not None:
                odesc[cur + j].wait()
        ta, tbuf = tb[cur], tb[cur + 1]
        oa, obuf = ob[cur], ob[cur + 1]

        @plsc.parallel_loop(0, CELLS, 16, unroll=8)
        def _gather(i, ta=ta, tbuf=tbuf, oa=oa, obuf=obuf):
            iv = inv_v[pl.ds(i, 16)]
            oa[pl.ds(i, 16)] = plsc.load_gather(ta, [iv])
            obuf[pl.ds(i, 16)] = plsc.load_gather(tbuf, [iv])

        for j in (0, 1):
            odesc[cur + j] = pltpu.async_copy(
                ob[cur + j], out_hbm.at[pl.ds((2 * k + j) * S + lo, CELLS)],
                osem[cur + j])
        tdesc = tnext

    for d in odesc:
        d.wait()


def kernel(pillar_features, coords):
    tbl, idx2 = _prep(pillar_features, coords)
    out = _sc_scatter(tbl, idx2)
    return out.reshape(1, C, NXG, NXG)


# speedup vs baseline: 2.2755x; 1.3372x over previous
import functools

import jax
import jax.numpy as jnp
from jax import lax
from jax.experimental import pallas as pl
from jax.experimental.pallas import tpu as pltpu
from jax.experimental.pallas import tpu_sc as plsc

C = 64
P = 20000
NXG = 512
S = 262144
P_PAD = 20016
IDX_PAD = 20480
SENT_IDX = 1 << 22
NC, NS = 2, 16
NW = NC * NS
CELLS = S // NW
G_CELL = CELLS // 16
ICHUNK = 4096
NCHUNK = IDX_PAD // ICHUNK
IROWS = ICHUNK // 128


def _prep_body(feat_ref, coords_ref, tbl_ref, idx_ref):
    x = feat_ref[...]
    xp = jnp.concatenate(
        [x, jnp.zeros((P_PAD - P, C), jnp.float32)], axis=0)
    tbl_ref[...] = xp.T
    cr = coords_ref[...]
    iv = cr[:, 1] + cr[:, 2] * NXG + cr[:, 3]
    idx_ref[...] = jnp.concatenate(
        [iv, jnp.full((IDX_PAD - P,), SENT_IDX, jnp.int32)]).reshape(
            IDX_PAD // 128, 128)


_prep = pl.pallas_call(
    _prep_body,
    compiler_params=pltpu.CompilerParams(vmem_limit_bytes=100 * 2**20),
    out_shape=(
        jax.ShapeDtypeStruct((C, P_PAD), jnp.float32),
        jax.ShapeDtypeStruct((IDX_PAD // 128, 128), jnp.int32),
    ),
)

_mesh = plsc.VectorSubcoreMesh(
    core_axis_name="c", subcore_axis_name="s", num_cores=NC, num_subcores=NS)


@functools.partial(
    pl.kernel,
    mesh=_mesh,
    compiler_params=pltpu.CompilerParams(needs_layout_passes=False),
    out_type=jax.ShapeDtypeStruct((1, C, NXG, NXG), jnp.float32),
    scratch_types=[
        [pltpu.VMEM((IROWS, 128), jnp.int32)] * 2,
        pltpu.VMEM((CELLS,), jnp.int32),
        [pltpu.VMEM((P_PAD,), jnp.float32)] * 4,
        [pltpu.VMEM((CELLS // NXG, NXG), jnp.float32)] * 4,
        [pltpu.SemaphoreType.DMA] * 2,
        [pltpu.SemaphoreType.DMA] * 4,
        [pltpu.SemaphoreType.DMA] * 4,
    ],
)
def _sc_scatter(tbl_hbm, idx_hbm, out_hbm, ib, inv_v, tb, ob, isem, tsem, osem):
    wid = lax.axis_index("s") * NC + lax.axis_index("c")
    lo = wid * CELLS
    y0 = wid * (CELLS // NXG)

    tdesc = [
        pltpu.async_copy(tbl_hbm.at[c], tb[c], tsem[c]) for c in (0, 1)
    ]
    idesc = pltpu.async_copy(idx_hbm.at[pl.ds(0, IROWS)], ib[0], isem[0])

    sent = jnp.full((16,), P, jnp.int32)

    @plsc.parallel_loop(0, CELLS, 16, unroll=8)
    def _init(i):
        inv_v[pl.ds(i, 16)] = sent

    lane = lax.broadcasted_iota(jnp.int32, (16,), 0)
    for ch in range(NCHUNK):
        nxt = pltpu.async_copy(
            idx_hbm.at[pl.ds((ch + 1) * IROWS, IROWS)],
            ib[(ch + 1) % 2], isem[(ch + 1) % 2]) if ch + 1 < NCHUNK else None
        idesc.wait()
        buf = ib[ch % 2]
        base = ch * ICHUNK

        def scan_body(g, carry, buf=buf, base=base):
            rel = buf[g // 8, pl.ds((g % 8) * 16, 16)] - lo
            m = (rel >= 0) & (rel < CELLS)
            rel_safe = jnp.where(m, rel, 0)
            plsc.store_scatter(
                inv_v, [rel_safe], base + g * 16 + lane, mask=m)
            return carry

        lax.fori_loop(0, ICHUNK // 16, scan_body, 0)
        idesc = nxt

    odesc = [None] * 4
    for k in range(C // 2):
        cur, nxt = 2 * (k % 2), 2 * ((k + 1) % 2)
        if k + 1 < C // 2:
            tnext = [
                pltpu.async_copy(
                    tbl_hbm.at[2 * (k + 1) + j], tb[nxt + j], tsem[nxt + j])
                for j in (0, 1)
            ]
        else:
            tnext = None
        tdesc[0].wait()
        tdesc[1].wait()
        for j in (0, 1):
            if odesc[cur + j] is not None:
                odesc[cur + j].wait()
        ta, tbuf = tb[cur], tb[cur + 1]
        oa, obuf = ob[cur], ob[cur + 1]

        @plsc.parallel_loop(0, CELLS, 16, unroll=8)
        def _gather(i, ta=ta, tbuf=tbuf, oa=oa, obuf=obuf):
            iv = inv_v[pl.ds(i, 16)]
            r, cc = i // NXG, i % NXG
            oa[r, pl.ds(cc, 16)] = plsc.load_gather(ta, [iv])
            obuf[r, pl.ds(cc, 16)] = plsc.load_gather(tbuf, [iv])

        for j in (0, 1):
            odesc[cur + j] = pltpu.async_copy(
                ob[cur + j],
                out_hbm.at[0, 2 * k + j, pl.ds(y0, CELLS // NXG)],
                osem[cur + j])
        tdesc = tnext

    for d in odesc:
        d.wait()


def kernel(pillar_features, coords):
    tbl, idx2 = _prep(pillar_features, coords)
    return _sc_scatter(tbl, idx2)


# speedup vs baseline: 3.0281x; 1.3308x over previous
import functools

import jax
import jax.numpy as jnp
from jax import lax
from jax.experimental import pallas as pl
from jax.experimental.pallas import tpu as pltpu
from jax.experimental.pallas import tpu_sc as plsc

C = 64
P = 20000
NXG = 512
S = 262144
P_PAD = 20008
IDX_PAD = 20480
SENT_IDX = 1 << 22
NC, NS = 2, 16
NW = NC * NS
CELLS = S // NW
ICHUNK = 2048
NCHUNK = IDX_PAD // ICHUNK
IROWS = ICHUNK // 128
CPT = C // NS
HC = 4096
NHC = S // NC // HC


def _prep_body(feat_ref, coords_ref, tbl_ref, idx_ref):
    x = feat_ref[...]
    xp = jnp.concatenate(
        [x, jnp.zeros((P_PAD - P, C), jnp.float32)], axis=0)
    tbl_ref[...] = xp.T
    cr = coords_ref[...]
    iv = cr[:, 1] + cr[:, 2] * NXG + cr[:, 3]
    idx_ref[...] = jnp.concatenate(
        [iv, jnp.full((IDX_PAD - P,), SENT_IDX, jnp.int32)]).reshape(
            IDX_PAD // 128, 128)


_prep = pl.pallas_call(
    _prep_body,
    compiler_params=pltpu.CompilerParams(vmem_limit_bytes=100 * 2**20),
    out_shape=(
        jax.ShapeDtypeStruct((C, P_PAD), jnp.float32),
        jax.ShapeDtypeStruct((IDX_PAD // 128, 128), jnp.int32),
    ),
)

_mesh = plsc.VectorSubcoreMesh(
    core_axis_name="c", subcore_axis_name="s", num_cores=NC, num_subcores=NS)


@functools.partial(
    pl.kernel,
    mesh=_mesh,
    compiler_params=pltpu.CompilerParams(needs_layout_passes=False),
    out_type=(
        jax.ShapeDtypeStruct((1, C, NXG, NXG), jnp.float32),
        jax.ShapeDtypeStruct((S,), jnp.int32),
    ),
    scratch_types=[
        [pltpu.VMEM((IROWS, 128), jnp.int32)] * 2,
        pltpu.VMEM((CELLS,), jnp.int32),
        [pltpu.VMEM((P_PAD,), jnp.float32)] * CPT,
        pltpu.VMEM((HC,), jnp.int32),
        [pltpu.VMEM((CPT, HC // NXG, NXG), jnp.float32)] * 2,
        [pltpu.SemaphoreType.DMA] * 2,
        [pltpu.SemaphoreType.DMA] * CPT,
        pltpu.SemaphoreType.DMA,
        [pltpu.SemaphoreType.DMA] * 2,
    ],
)
def _sc_scatter(tbl_hbm, idx_hbm, out_hbm, inv_hbm, ib, inv_v, tb, ivc, ob,
                isem, tsem, ivsem, osem):
    cid = lax.axis_index("c")
    sid = lax.axis_index("s")
    lo = (cid * NS + sid) * CELLS
    c0 = sid * CPT
    half_lo = cid * (S // NC)

    tdesc = [
        pltpu.async_copy(tbl_hbm.at[c0 + j], tb[j], tsem[j])
        for j in range(CPT)
    ]
    idesc = pltpu.async_copy(idx_hbm.at[pl.ds(0, IROWS)], ib[0], isem[0])

    sent = jnp.full((16,), P, jnp.int32)

    @plsc.parallel_loop(0, CELLS, 16, unroll=8)
    def _init(i):
        inv_v[pl.ds(i, 16)] = sent

    lane = lax.broadcasted_iota(jnp.int32, (16,), 0)
    for ch in range(NCHUNK):
        nxt = pltpu.async_copy(
            idx_hbm.at[pl.ds((ch + 1) * IROWS, IROWS)],
            ib[(ch + 1) % 2], isem[(ch + 1) % 2]) if ch + 1 < NCHUNK else None
        idesc.wait()
        buf = ib[ch % 2]
        base = ch * ICHUNK

        def scan_body(g, carry, buf=buf, base=base):
            rel = buf[g // 8, pl.ds((g % 8) * 16, 16)] - lo
            m = (rel >= 0) & (rel < CELLS)
            rel_safe = jnp.where(m, rel, 0)
            plsc.store_scatter(
                inv_v, [rel_safe], base + g * 16 + lane, mask=m)
            return carry

        lax.fori_loop(0, ICHUNK // 16, scan_body, 0)
        idesc = nxt

    pltpu.sync_copy(inv_v, inv_hbm.at[pl.ds(lo, CELLS)])
    plsc.subcore_barrier()

    for d in tdesc:
        d.wait()

    ivdesc = pltpu.async_copy(inv_hbm.at[pl.ds(half_lo, HC)], ivc, ivsem)
    odesc = [None, None]
    for k in range(NHC):
        ivdesc.wait()
        if odesc[k % 2] is not None:
            odesc[k % 2].wait()
        stg = ob[k % 2]

        @plsc.parallel_loop(0, HC, 16, unroll=8)
        def _gather(i, stg=stg):
            iv = ivc[pl.ds(i, 16)]
            r, x = i // NXG, i % NXG
            stg[0, r, pl.ds(x, 16)] = plsc.load_gather(tb[0], [iv])
            stg[1, r, pl.ds(x, 16)] = plsc.load_gather(tb[1], [iv])
            stg[2, r, pl.ds(x, 16)] = plsc.load_gather(tb[2], [iv])
            stg[3, r, pl.ds(x, 16)] = plsc.load_gather(tb[3], [iv])

        if k + 1 < NHC:
            ivdesc = pltpu.async_copy(
                inv_hbm.at[pl.ds(half_lo + (k + 1) * HC, HC)], ivc, ivsem)
        y = cid * (NXG // NC) + k * (HC // NXG)
        odesc[k % 2] = pltpu.async_copy(
            stg,
            out_hbm.at[0, pl.ds(c0, CPT), pl.ds(y, HC // NXG)],
            osem[k % 2])

    odesc[0].wait()
    odesc[1].wait()


def kernel(pillar_features, coords):
    tbl, idx2 = _prep(pillar_features, coords)
    out, _ = _sc_scatter(tbl, idx2)
    return out


# speedup vs baseline: 3.5600x; 1.1757x over previous
import functools

import jax
import jax.numpy as jnp
from jax import lax
from jax.experimental import pallas as pl
from jax.experimental.pallas import tpu as pltpu
from jax.experimental.pallas import tpu_sc as plsc

C = 64
P = 20000
NXG = 512
S = 262144
P_PAD = 20008
NC, NS = 2, 16
NW = NC * NS
CELLS = S // NW
CCH = 2048
CPT = C // NS
HC = 4096
NHC = S // NC // HC

_mesh = plsc.VectorSubcoreMesh(
    core_axis_name="c", subcore_axis_name="s", num_cores=NC, num_subcores=NS)


IDX_PAD = 20480
SENT_IDX = 1 << 22
ICHUNK = 2048
NCHUNK = IDX_PAD // ICHUNK
IROWS = ICHUNK // 128


def _prep_idx_body(coords_ref, idx_ref):
    cr = coords_ref[...]
    iv = cr[:, 1] + cr[:, 2] * NXG + cr[:, 3]
    idx_ref[...] = jnp.concatenate(
        [iv, jnp.full((IDX_PAD - P,), SENT_IDX, jnp.int32)]).reshape(
            IDX_PAD // 128, 128)


_prep_idx = pl.pallas_call(
    _prep_idx_body,
    out_shape=jax.ShapeDtypeStruct((IDX_PAD // 128, 128), jnp.int32),
)


def _prep_tbl_body(feat_ref, tbl_ref):
    x = feat_ref[...]
    xp = jnp.concatenate(
        [x, jnp.zeros((P_PAD - P, C), jnp.float32)], axis=0)
    tbl_ref[...] = xp.T


_prep_tbl = pl.pallas_call(
    _prep_tbl_body,
    compiler_params=pltpu.CompilerParams(vmem_limit_bytes=100 * 2**20),
    out_shape=jax.ShapeDtypeStruct((C, P_PAD), jnp.float32),
)


@functools.partial(
    pl.kernel,
    mesh=_mesh,
    compiler_params=pltpu.CompilerParams(needs_layout_passes=False),
    out_type=jax.ShapeDtypeStruct((S,), jnp.int32),
    scratch_types=[
        [pltpu.VMEM((IROWS, 128), jnp.int32)] * 2,
        pltpu.VMEM((CELLS,), jnp.int32),
        [pltpu.SemaphoreType.DMA] * 2,
    ],
)
def _sc_invmap(idx_hbm, inv_hbm, ib, inv_v, isem):
    cid = lax.axis_index("c")
    sid = lax.axis_index("s")
    lo = (cid * NS + sid) * CELLS

    idesc = pltpu.async_copy(idx_hbm.at[pl.ds(0, IROWS)], ib[0], isem[0])

    sent = jnp.full((16,), P, jnp.int32)

    @plsc.parallel_loop(0, CELLS, 16, unroll=8)
    def _init(i):
        inv_v[pl.ds(i, 16)] = sent

    lane = lax.broadcasted_iota(jnp.int32, (16,), 0)
    for ch in range(NCHUNK):
        nxt = pltpu.async_copy(
            idx_hbm.at[pl.ds((ch + 1) * IROWS, IROWS)],
            ib[(ch + 1) % 2], isem[(ch + 1) % 2]) if ch + 1 < NCHUNK else None
        idesc.wait()
        buf = ib[ch % 2]
        base = ch * ICHUNK

        def scan_body(g, carry, buf=buf, base=base):
            rel = buf[g // 8, pl.ds((g % 8) * 16, 16)] - lo
            m = (rel >= 0) & (rel < CELLS)
            rel_safe = jnp.where(m, rel, 0)
            plsc.store_scatter(
                inv_v, [rel_safe], base + g * 16 + lane, mask=m)
            return carry

        lax.fori_loop(0, ICHUNK // 16, scan_body, 0)
        idesc = nxt

    pltpu.sync_copy(inv_v, inv_hbm.at[pl.ds(lo, CELLS)])


@functools.partial(
    pl.kernel,
    mesh=_mesh,
    compiler_params=pltpu.CompilerParams(needs_layout_passes=False),
    out_type=jax.ShapeDtypeStruct((1, C, NXG, NXG), jnp.float32),
    scratch_types=[
        [pltpu.VMEM((P_PAD,), jnp.float32)] * CPT,
        [pltpu.VMEM((HC,), jnp.int32)] * 2,
        [pltpu.VMEM((CPT, HC // NXG, NXG), jnp.float32)] * 2,
        [pltpu.SemaphoreType.DMA] * CPT,
        [pltpu.SemaphoreType.DMA] * 2,
        [pltpu.SemaphoreType.DMA] * 2,
    ],
)
def _sc_compose(tbl_hbm, inv_hbm, out_hbm, tb, ivc, ob, tsem, ivsem, osem):
    cid = lax.axis_index("c")
    sid = lax.axis_index("s")
    c0 = sid * CPT
    half_lo = cid * (S // NC)

    tdesc = [
        pltpu.async_copy(tbl_hbm.at[c0 + j], tb[j], tsem[j])
        for j in range(CPT)
    ]
    ivdesc = pltpu.async_copy(
        inv_hbm.at[pl.ds(pl.multiple_of(half_lo, HC), HC)], ivc[0], ivsem[0])
    for d in tdesc:
        d.wait()

    odesc = [None, None]
    for k in range(NHC):
        if k + 1 < NHC:
            ivnext = pltpu.async_copy(
                inv_hbm.at[pl.ds(pl.multiple_of(half_lo + (k + 1) * HC, HC),
                                 HC)],
                ivc[(k + 1) % 2], ivsem[(k + 1) % 2])
        else:
            ivnext = None
        ivdesc.wait()
        if odesc[k % 2] is not None:
            odesc[k % 2].wait()
        stg = ob[k % 2]
        ivb = ivc[k % 2]

        @plsc.parallel_loop(0, HC, 16, unroll=8)
        def _gather(i, stg=stg, ivb=ivb):
            iv = ivb[pl.ds(i, 16)]
            r, x = i // NXG, i % NXG
            stg[0, r, pl.ds(x, 16)] = plsc.load_gather(tb[0], [iv])
            stg[1, r, pl.ds(x, 16)] = plsc.load_gather(tb[1], [iv])
            stg[2, r, pl.ds(x, 16)] = plsc.load_gather(tb[2], [iv])
            stg[3, r, pl.ds(x, 16)] = plsc.load_gather(tb[3], [iv])

        y = pl.multiple_of(cid * (NXG // NC) + k * (HC // NXG), HC // NXG)
        odesc[k % 2] = pltpu.async_copy(
            stg,
            out_hbm.at[0, pl.ds(c0, CPT), pl.ds(y, HC // NXG)],
            osem[k % 2])
        ivdesc = ivnext

    odesc[0].wait()
    odesc[1].wait()


def kernel(pillar_features, coords):
    idx2 = _prep_idx(coords)
    inv = _sc_invmap(idx2)
    tbl = _prep_tbl(pillar_features)
    return _sc_compose(tbl, inv)


# speedup vs baseline: 3.5693x; 1.0026x over previous
import functools

import jax
import jax.numpy as jnp
from jax import lax
from jax.experimental import pallas as pl
from jax.experimental.pallas import tpu as pltpu
from jax.experimental.pallas import tpu_sc as plsc

C = 64
P = 20000
NXG = 512
S = 262144
P_PAD = 20480
TBLK = P_PAD // 8
NC, NS = 2, 16
NW = NC * NS
CELLS = S // NW
CCH = 2048
CPT = C // NS
HC = 4096
NHC = S // NC // HC

_mesh = plsc.VectorSubcoreMesh(
    core_axis_name="c", subcore_axis_name="s", num_cores=NC, num_subcores=NS)


IDX_PAD = 20480
SENT_IDX = 1 << 22
ICHUNK = 2048
NCHUNK = IDX_PAD // ICHUNK
IROWS = ICHUNK // 128


def _prep_idx_body(coords_ref, idx_ref):
    cr = coords_ref[...]
    iv = cr[:, 1] + cr[:, 2] * NXG + cr[:, 3]
    idx_ref[...] = jnp.concatenate(
        [iv, jnp.full((IDX_PAD - P,), SENT_IDX, jnp.int32)]).reshape(
            IDX_PAD // 128, 128)


_prep_idx = pl.pallas_call(
    _prep_idx_body,
    out_shape=jax.ShapeDtypeStruct((IDX_PAD // 128, 128), jnp.int32),
)


def _prep_tbl_body(feat_ref, tbl_ref):
    i = pl.program_id(0)
    x = feat_ref[...]
    rows = i * TBLK + lax.broadcasted_iota(jnp.int32, (TBLK, 1), 0)
    x = jnp.where(rows < P, x, 0.0)
    tbl_ref[...] = x.T


_prep_tbl = pl.pallas_call(
    _prep_tbl_body,
    grid=(P_PAD // TBLK,),
    in_specs=[pl.BlockSpec((TBLK, C), lambda i: (i, 0))],
    out_specs=pl.BlockSpec((C, TBLK), lambda i: (0, i)),
    out_shape=jax.ShapeDtypeStruct((C, P_PAD), jnp.float32),
)


@functools.partial(
    pl.kernel,
    mesh=_mesh,
    compiler_params=pltpu.CompilerParams(needs_layout_passes=False),
    out_type=jax.ShapeDtypeStruct((S,), jnp.int32),
    scratch_types=[
        [pltpu.VMEM((IROWS, 128), jnp.int32)] * 2,
        pltpu.VMEM((CELLS,), jnp.int32),
        [pltpu.SemaphoreType.DMA] * 2,
    ],
)
def _sc_invmap(idx_hbm, inv_hbm, ib, inv_v, isem):
    cid = lax.axis_index("c")
    sid = lax.axis_index("s")
    lo = (cid * NS + sid) * CELLS

    idesc = pltpu.async_copy(idx_hbm.at[pl.ds(0, IROWS)], ib[0], isem[0])

    sent = jnp.full((16,), P, jnp.int32)

    @plsc.parallel_loop(0, CELLS, 16, unroll=8)
    def _init(i):
        inv_v[pl.ds(i, 16)] = sent

    lane = lax.broadcasted_iota(jnp.int32, (16,), 0)
    for ch in range(NCHUNK):
        nxt = pltpu.async_copy(
            idx_hbm.at[pl.ds((ch + 1) * IROWS, IROWS)],
            ib[(ch + 1) % 2], isem[(ch + 1) % 2]) if ch + 1 < NCHUNK else None
        idesc.wait()
        buf = ib[ch % 2]
        base = ch * ICHUNK

        def scan_body(g, carry, buf=buf, base=base):
            rel = buf[g // 8, pl.ds((g % 8) * 16, 16)] - lo
            m = (rel >= 0) & (rel < CELLS)
            rel_safe = jnp.where(m, rel, 0)
            plsc.store_scatter(
                inv_v, [rel_safe], base + g * 16 + lane, mask=m)
            return carry

        lax.fori_loop(0, ICHUNK // 16, scan_body, 0)
        idesc = nxt

    pltpu.sync_copy(inv_v, inv_hbm.at[pl.ds(lo, CELLS)])


@functools.partial(
    pl.kernel,
    mesh=_mesh,
    compiler_params=pltpu.CompilerParams(needs_layout_passes=False),
    out_type=jax.ShapeDtypeStruct((1, C, NXG, NXG), jnp.float32),
    scratch_types=[
        [pltpu.VMEM((P_PAD,), jnp.float32)] * CPT,
        [pltpu.VMEM((HC,), jnp.int32)] * 2,
        [pltpu.VMEM((CPT, HC // NXG, NXG), jnp.float32)] * 2,
        [pltpu.SemaphoreType.DMA] * CPT,
        [pltpu.SemaphoreType.DMA] * 2,
        [pltpu.SemaphoreType.DMA] * 2,
    ],
)
def _sc_compose(tbl_hbm, inv_hbm, out_hbm, tb, ivc, ob, tsem, ivsem, osem):
    cid = lax.axis_index("c")
    sid = lax.axis_index("s")
    c0 = sid * CPT
    half_lo = cid * (S // NC)

    tdesc = [
        pltpu.async_copy(tbl_hbm.at[c0 + j], tb[j], tsem[j])
        for j in range(CPT)
    ]
    ivdesc = pltpu.async_copy(
        inv_hbm.at[pl.ds(pl.multiple_of(half_lo, HC), HC)], ivc[0], ivsem[0])
    for d in tdesc:
        d.wait()

    odesc = [None, None]
    for k in range(NHC):
        if k + 1 < NHC:
            ivnext = pltpu.async_copy(
                inv_hbm.at[pl.ds(pl.multiple_of(half_lo + (k + 1) * HC, HC),
                                 HC)],
                ivc[(k + 1) % 2], ivsem[(k + 1) % 2])
        else:
            ivnext = None
        ivdesc.wait()
        if odesc[k % 2] is not None:
            odesc[k % 2].wait()
        stg = ob[k % 2]
        ivb = ivc[k % 2]

        @plsc.parallel_loop(0, HC, 16, unroll=8)
        def _gather(i, stg=stg, ivb=ivb):
            iv = ivb[pl.ds(i, 16)]
            r, x = i // NXG, i % NXG
            stg[0, r, pl.ds(x, 16)] = plsc.load_gather(tb[0], [iv])
            stg[1, r, pl.ds(x, 16)] = plsc.load_gather(tb[1], [iv])
            stg[2, r, pl.ds(x, 16)] = plsc.load_gather(tb[2], [iv])
            stg[3, r, pl.ds(x, 16)] = plsc.load_gather(tb[3], [iv])

        y = pl.multiple_of(cid * (NXG // NC) + k * (HC // NXG), HC // NXG)
        odesc[k % 2] = pltpu.async_copy(
            stg,
            out_hbm.at[0, pl.ds(c0, CPT), pl.ds(y, HC // NXG)],
            osem[k % 2])
        ivdesc = ivnext

    odesc[0].wait()
    odesc[1].wait()


def kernel(pillar_features, coords):
    idx2 = _prep_idx(coords)
    inv = _sc_invmap(idx2)
    tbl = _prep_tbl(pillar_features)
    return _sc_compose(tbl, inv)


# speedup vs baseline: 3.6175x; 1.0135x over previous
import functools

import jax
import jax.numpy as jnp
from jax import lax
from jax.experimental import pallas as pl
from jax.experimental.pallas import tpu as pltpu
from jax.experimental.pallas import tpu_sc as plsc

C = 64
P = 20000
NXG = 512
S = 262144
P_PAD = 20480
TBLK = P_PAD // 8
NC, NS = 2, 16
NW = NC * NS
CELLS = S // NW
CCH = 2048
CPT = C // NS
HC = 4096
NHC = S // NC // HC

_mesh = plsc.VectorSubcoreMesh(
    core_axis_name="c", subcore_axis_name="s", num_cores=NC, num_subcores=NS)


CROWS = P * 4 // 128
CRCH = 64


def _prep_tbl_body(feat_ref, tbl_ref):
    i = pl.program_id(0)
    x = feat_ref[...]
    rows = i * TBLK + lax.broadcasted_iota(jnp.int32, (TBLK, 1), 0)
    x = jnp.where(rows < P, x, 0.0)
    tbl_ref[...] = x.T


_prep_tbl = pl.pallas_call(
    _prep_tbl_body,
    grid=(P_PAD // TBLK,),
    in_specs=[pl.BlockSpec((TBLK, C), lambda i: (i, 0))],
    out_specs=pl.BlockSpec((C, TBLK), lambda i: (0, i)),
    out_shape=jax.ShapeDtypeStruct((C, P_PAD), jnp.float32),
)


@functools.partial(
    pl.kernel,
    mesh=_mesh,
    compiler_params=pltpu.CompilerParams(needs_layout_passes=False),
    out_type=jax.ShapeDtypeStruct((S,), jnp.int32),
    scratch_types=[
        [pltpu.VMEM((CRCH, 128), jnp.int32)] * 2,
        pltpu.VMEM((CELLS,), jnp.int32),
        [pltpu.SemaphoreType.DMA] * 2,
    ],
)
def _sc_invmap(coords_hbm, inv_hbm, cb, inv_v, csem):
    cid = lax.axis_index("c")
    sid = lax.axis_index("s")
    lo = (cid * NS + sid) * CELLS

    chunks = [CRCH] * (CROWS // CRCH) + (
        [CROWS % CRCH] if CROWS % CRCH else [])
    cdesc = pltpu.async_copy(
        coords_hbm.at[pl.ds(0, chunks[0])], cb[0].at[pl.ds(0, chunks[0])],
        csem[0])

    sent = jnp.full((16,), P, jnp.int32)

    @plsc.parallel_loop(0, CELLS, 16, unroll=8)
    def _init(i):
        inv_v[pl.ds(i, 16)] = sent

    lane = lax.broadcasted_iota(jnp.int32, (16,), 0)
    pbase = 0
    for ci, chn in enumerate(chunks):
        if ci + 1 < len(chunks):
            nxt = pltpu.async_copy(
                coords_hbm.at[pl.ds((ci + 1) * CRCH, chunks[ci + 1])],
                cb[(ci + 1) % 2].at[pl.ds(0, chunks[ci + 1])],
                csem[(ci + 1) % 2])
        else:
            nxt = None
        cdesc.wait()
        buf = cb[ci % 2]

        def cbody(g, carry, buf=buf, pbase=pbase):
            flat = (g * 16 + lane) * 4 + 1
            c1 = plsc.load_gather(buf, [flat // 128, flat % 128])
            c2 = plsc.load_gather(buf, [flat // 128, flat % 128 + 1])
            c3 = plsc.load_gather(buf, [flat // 128, flat % 128 + 2])
            rel = c1 + c2 * NXG + c3 - lo
            m = (rel >= 0) & (rel < CELLS)
            rel_safe = jnp.where(m, rel, 0)
            plsc.store_scatter(
                inv_v, [rel_safe], pbase + g * 16 + lane, mask=m)
            return carry

        lax.fori_loop(0, chn * 2, cbody, 0)
        pbase += chn * 32
        cdesc = nxt

    pltpu.sync_copy(inv_v, inv_hbm.at[pl.ds(lo, CELLS)])


@functools.partial(
    pl.kernel,
    mesh=_mesh,
    compiler_params=pltpu.CompilerParams(needs_layout_passes=False),
    out_type=jax.ShapeDtypeStruct((1, C, NXG, NXG), jnp.float32),
    scratch_types=[
        [pltpu.VMEM((P_PAD,), jnp.float32)] * CPT,
        [pltpu.VMEM((HC,), jnp.int32)] * 2,
        [pltpu.VMEM((CPT, HC // NXG, NXG), jnp.float32)] * 2,
        [pltpu.SemaphoreType.DMA] * CPT,
        [pltpu.SemaphoreType.DMA] * 2,
        [pltpu.SemaphoreType.DMA] * 2,
    ],
)
def _sc_compose(tbl_hbm, inv_hbm, out_hbm, tb, ivc, ob, tsem, ivsem, osem):
    cid = lax.axis_index("c")
    sid = lax.axis_index("s")
    c0 = sid * CPT
    half_lo = cid * (S // NC)

    tdesc = [
        pltpu.async_copy(tbl_hbm.at[c0 + j], tb[j], tsem[j])
        for j in range(CPT)
    ]
    ivdesc = pltpu.async_copy(
        inv_hbm.at[pl.ds(pl.multiple_of(half_lo, HC), HC)], ivc[0], ivsem[0])
    for d in tdesc:
        d.wait()

    odesc = [None, None]
    for k in range(NHC):
        if k + 1 < NHC:
            ivnext = pltpu.async_copy(
                inv_hbm.at[pl.ds(pl.multiple_of(half_lo + (k + 1) * HC, HC),
                                 HC)],
                ivc[(k + 1) % 2], ivsem[(k + 1) % 2])
        else:
            ivnext = None
        ivdesc.wait()
        if odesc[k % 2] is not None:
            odesc[k % 2].wait()
        stg = ob[k % 2]
        ivb = ivc[k % 2]

        @plsc.parallel_loop(0, HC, 16, unroll=8)
        def _gather(i, stg=stg, ivb=ivb):
            iv = ivb[pl.ds(i, 16)]
            r, x = i // NXG, i % NXG
            stg[0, r, pl.ds(x, 16)] = plsc.load_gather(tb[0], [iv])
            stg[1, r, pl.ds(x, 16)] = plsc.load_gather(tb[1], [iv])
            stg[2, r, pl.ds(x, 16)] = plsc.load_gather(tb[2], [iv])
            stg[3, r, pl.ds(x, 16)] = plsc.load_gather(tb[3], [iv])

        y = pl.multiple_of(cid * (NXG // NC) + k * (HC // NXG), HC // NXG)
        odesc[k % 2] = pltpu.async_copy(
            stg,
            out_hbm.at[0, pl.ds(c0, CPT), pl.ds(y, HC // NXG)],
            osem[k % 2])
        ivdesc = ivnext

    odesc[0].wait()
    odesc[1].wait()


def kernel(pillar_features, coords):
    inv = _sc_invmap(coords.reshape(CROWS, 128))
    tbl = _prep_tbl(pillar_features)
    return _sc_compose(tbl, inv)
